# baseline (speedup 1.0000x reference)
import functools

import numpy as np
import jax
import jax.numpy as jnp
from jax import lax
from jax.experimental import pallas as pl
from jax.experimental.pallas import tpu as pltpu
from jax.experimental.pallas import tpu_sc as plsc

B = 16
H = 512
W = 512
NSEG = 1000
NC = 2
NS = 16
NW = NC * NS
L = 16
SLABW = 1024
SLAB = SLABW * SLABW
RANGE = SLAB // NW
CH = 16
N_CHUNK = H // CH
VPR = W // L
NV = CH * VPR
NLAB = 16000

_GDN = lax.GatherDimensionNumbers(
    offset_dims=(), collapsed_slice_dims=(0,), start_index_map=(0,)
)


def _rot1(v, perm2d):
    return lax.gather(
        v, perm2d, _GDN, (1,), mode=lax.GatherScatterMode.PROMISE_IN_BOUNDS
    )


_INV1000 = np.float32(0.001000000047497451)


def _div1000(x):
    return (x.astype(jnp.float32) * _INV1000).astype(jnp.int32)


def _round_bf16(x_i32):
    u = plsc.bitcast(x_i32.astype(jnp.float32), jnp.int32)
    t = u + 0x7FFF + ((u >> 16) & 1)
    t = t & jnp.int32(-65536)
    return plsc.bitcast(t, jnp.float32).astype(jnp.int32)


def _sc_body(seg_hbm, out_hbm, chunk_v, hist_v, lut_v):
    c = lax.axis_index("c")
    s = lax.axis_index("s")
    wid = s * NC + c
    lo = wid * RANGE
    ones = jnp.ones((L,), jnp.float32)
    zeros = jnp.zeros((L,), jnp.float32)
    lane = lax.iota(jnp.int32, L)
    perm2d = ((lane + 1) & (L - 1))[:, None]
    lane15 = lane == L - 1
    million = jnp.int32(1_000_000)

    def lut_body(k, carry):
        x = k * L + lane
        xr = _round_bf16(x)
        eb1000 = _div1000(xr) * NSEG
        lut_v[pl.ds(pl.multiple_of(k * L, L), L)] = (eb1000 << 14) | xr
        return carry

    lax.fori_loop(0, NLAB // L, lut_body, 0)

    def zero_slot(slot):
        def zbody(k, carry):
            hist_v[pl.ds(slot * RANGE + k * L, L)] = zeros
            return carry

        lax.fori_loop(0, RANGE // L, zbody, 0)

    for slot in range(3):
        zero_slot(slot)

    def batch_body(b, carry):
        off_b = NSEG * b
        slot_prev = (b + 2) % 3
        base_prev = slot_prev * RANGE
        base_cur = (b % 3) * RANGE
        base_next = ((b + 1) % 3) * RANGE
        off_bm1_1000 = NSEG * (b - 1)

        def chunk_body(ci, carry):
            pltpu.sync_copy(seg_hbm.at[b, pl.ds(ci * CH, CH), :], chunk_v)
            p0 = plsc.load_gather(lut_v, [chunk_v[0, pl.ds(0, L)] + off_b])

            def inner(t, p_cur):
                tn = jnp.minimum(t + 1, NV - 1)
                rn = tn >> 5
                jn = tn & (VPR - 1)
                raw_n = chunk_v[rn, pl.ds(pl.multiple_of(jn * L, L), L)]
                p_next = plsc.load_gather(lut_v, [raw_n + off_b])
                p_d = jnp.where(lane15, _rot1(p_next, perm2d), _rot1(p_cur, perm2d))
                valid = (p_cur != p_d) & ~(lane15 & ((t & (VPR - 1)) == VPR - 1))
                x1 = p_cur & 0x3FFF
                x2 = p_d & 0x3FFF
                eb1000 = p_cur >> 14
                rem0 = (x1 - eb1000) * NSEG + (x2 - eb1000)
                neg = rem0 < 0
                big = rem0 >= million
                rem = rem0 + jnp.where(neg, million, jnp.where(big, -million, 0))
                fi = _div1000(rem)
                fj = rem - fi * NSEG
                reml = (fi << 10) | fj
                d1000 = (eb1000 - off_bm1_1000) + jnp.where(
                    neg, -NSEG, jnp.where(big, NSEG, 0)
                )
                base = jnp.where(
                    d1000 == 0,
                    base_prev,
                    jnp.where(d1000 == NSEG, base_cur, base_next),
                )
                u = reml - lo
                m = valid & (u >= 0) & (u < RANGE)
                plsc.addupdate_scatter(hist_v, [base + u], ones, mask=m)
                return p_next

            lax.fori_loop(0, NV, inner, p0)
            return carry

        lax.fori_loop(0, N_CHUNK, chunk_body, 0)

        @pl.when(b >= 1)
        def _flush():
            off = pl.multiple_of((b - 1) * SLAB + lo, 8)
            pltpu.sync_copy(
                hist_v.at[pl.ds(base_prev, RANGE)], out_hbm.at[pl.ds(off, RANGE)]
            )
            zero_slot(slot_prev)

        return carry

    lax.fori_loop(0, B, batch_body, 0, unroll=3)
    off = pl.multiple_of((B - 1) * SLAB + lo, 8)
    pltpu.sync_copy(hist_v.at[pl.ds(0, RANGE)], out_hbm.at[pl.ds(off, RANGE)])


def _sc_histogram(segments):
    mesh = plsc.VectorSubcoreMesh(
        core_axis_name="c", subcore_axis_name="s", num_cores=NC, num_subcores=NS
    )
    return pl.kernel(
        _sc_body,
        out_type=jax.ShapeDtypeStruct((B * SLAB,), jnp.float32),
        mesh=mesh,
        scratch_types=[
            pltpu.VMEM((CH, W), jnp.int32),
            pltpu.VMEM((3 * RANGE,), jnp.float32),
            pltpu.VMEM((NLAB,), jnp.int32),
        ],
        compiler_params=pltpu.CompilerParams(needs_layout_passes=False),
    )(segments)


def _sym_body(x_ref, o_ref):
    x = x_ref[0]
    y = x + x.T
    o_ref[0] = y[:NSEG, :NSEG]


def _symmetrize(cnt):
    return pl.pallas_call(
        _sym_body,
        grid=(B,),
        in_specs=[pl.BlockSpec((1, SLABW, SLABW), lambda b: (b, 0, 0))],
        out_specs=pl.BlockSpec((1, NSEG, NSEG), lambda b: (b, 0, 0)),
        out_shape=jax.ShapeDtypeStruct((B, NSEG, NSEG), jnp.float32),
    )(cnt)


@jax.jit
def kernel(segments):
    out_flat = _sc_histogram(segments)
    return _symmetrize(out_flat.reshape(B, SLABW, SLABW))


# speedup vs baseline: 1.0050x; 1.0050x over previous
import functools

import numpy as np
import jax
import jax.numpy as jnp
from jax import lax
from jax.experimental import pallas as pl
from jax.experimental.pallas import tpu as pltpu
from jax.experimental.pallas import tpu_sc as plsc

B = 16
H = 512
W = 512
NSEG = 1000
NC = 2
NS = 16
NW = NC * NS
L = 16
SLABW = 1024
SLAB = SLABW * SLABW
RANGE = SLAB // NW
CH = 32
N_CHUNK = H // CH
VPR = W // L
NV = CH * VPR
NLAB = 16000

_GDN = lax.GatherDimensionNumbers(
    offset_dims=(), collapsed_slice_dims=(0,), start_index_map=(0,)
)


def _rot1(v, perm2d):
    return lax.gather(
        v, perm2d, _GDN, (1,), mode=lax.GatherScatterMode.PROMISE_IN_BOUNDS
    )


_INV1000 = np.float32(0.001000000047497451)


def _div1000(x):
    return (x.astype(jnp.float32) * _INV1000).astype(jnp.int32)


def _round_bf16(x_i32):
    u = plsc.bitcast(x_i32.astype(jnp.float32), jnp.int32)
    t = u + 0x7FFF + ((u >> 16) & 1)
    t = t & jnp.int32(-65536)
    return plsc.bitcast(t, jnp.float32).astype(jnp.int32)


def _sc_body(seg_hbm, out_hbm, chunk_v, hist_v):
    c = lax.axis_index("c")
    s = lax.axis_index("s")
    wid = s * NC + c
    lo = wid * RANGE
    ones = jnp.ones((L,), jnp.float32)
    zeros = jnp.zeros((L,), jnp.float32)
    lane = lax.iota(jnp.int32, L)
    perm2d = ((lane + 1) & (L - 1))[:, None]
    lane15 = lane == L - 1
    million = jnp.int32(1_000_000)

    def zero_slot(slot):
        def zbody(k, carry):
            hist_v[pl.ds(slot * RANGE + k * L, L)] = zeros
            return carry

        lax.fori_loop(0, RANGE // L, zbody, 0)

    for slot in range(3):
        zero_slot(slot)

    def batch_body(b, carry):
        off_b = NSEG * b
        slot_prev = (b + 2) % 3
        base_prev = slot_prev * RANGE
        base_cur = (b % 3) * RANGE
        base_next = ((b + 1) % 3) * RANGE
        off_bm1_1000 = NSEG * (b - 1)

        def chunk_body(ci, carry):
            pltpu.sync_copy(seg_hbm.at[b, pl.ds(ci * CH, CH), :], chunk_v)

            def prep(t, carry):
                r = t >> 5
                j = t & (VPR - 1)
                raw = chunk_v[r, pl.ds(pl.multiple_of(j * L, L), L)]
                xr = _round_bf16(raw + off_b)
                eb1000 = _div1000(xr) * NSEG
                chunk_v[r, pl.ds(pl.multiple_of(j * L, L), L)] = (eb1000 << 14) | xr
                return carry

            lax.fori_loop(0, NV, prep, 0, unroll=4)

            def inner(t, carry):
                r = t >> 5
                j = t & (VPR - 1)
                p_cur = chunk_v[r, pl.ds(pl.multiple_of(j * L, L), L)]
                tn = jnp.minimum(t + 1, NV - 1)
                rn = tn >> 5
                jn = tn & (VPR - 1)
                p_next = chunk_v[rn, pl.ds(pl.multiple_of(jn * L, L), L)]
                p_d = jnp.where(lane15, _rot1(p_next, perm2d), _rot1(p_cur, perm2d))
                valid = (p_cur != p_d) & ~(lane15 & ((t & (VPR - 1)) == VPR - 1))
                x1 = p_cur & 0x3FFF
                x2 = p_d & 0x3FFF
                eb1000 = p_cur >> 14
                rem0 = (x1 - eb1000) * NSEG + (x2 - eb1000)
                neg = rem0 < 0
                big = rem0 >= million
                rem = rem0 + jnp.where(neg, million, jnp.where(big, -million, 0))
                fi = _div1000(rem)
                fj = rem - fi * NSEG
                reml = (fi << 10) | fj
                d1000 = (eb1000 - off_bm1_1000) + jnp.where(
                    neg, -NSEG, jnp.where(big, NSEG, 0)
                )
                base = jnp.where(
                    d1000 == 0,
                    base_prev,
                    jnp.where(d1000 == NSEG, base_cur, base_next),
                )
                u = reml - lo
                m = valid & (u >= 0) & (u < RANGE)
                plsc.addupdate_scatter(hist_v, [base + u], ones, mask=m)
                return carry

            lax.fori_loop(0, NV, inner, 0, unroll=4)
            return carry

        lax.fori_loop(0, N_CHUNK, chunk_body, 0)

        @pl.when(b >= 1)
        def _flush():
            off = pl.multiple_of((b - 1) * SLAB + lo, 8)
            pltpu.sync_copy(
                hist_v.at[pl.ds(base_prev, RANGE)], out_hbm.at[pl.ds(off, RANGE)]
            )
            zero_slot(slot_prev)

        return carry

    lax.fori_loop(0, B, batch_body, 0, unroll=3)
    off = pl.multiple_of((B - 1) * SLAB + lo, 8)
    pltpu.sync_copy(hist_v.at[pl.ds(0, RANGE)], out_hbm.at[pl.ds(off, RANGE)])


def _sc_histogram(segments):
    mesh = plsc.VectorSubcoreMesh(
        core_axis_name="c", subcore_axis_name="s", num_cores=NC, num_subcores=NS
    )
    return pl.kernel(
        _sc_body,
        out_type=jax.ShapeDtypeStruct((B * SLAB,), jnp.float32),
        mesh=mesh,
        scratch_types=[
            pltpu.VMEM((CH, W), jnp.int32),
            pltpu.VMEM((3 * RANGE,), jnp.float32),
        ],
        compiler_params=pltpu.CompilerParams(needs_layout_passes=False),
    )(segments)


def _sym_body(x_ref, o_ref):
    x = x_ref[0]
    y = x + x.T
    o_ref[0] = y[:NSEG, :NSEG]


def _symmetrize(cnt):
    return pl.pallas_call(
        _sym_body,
        grid=(B,),
        in_specs=[pl.BlockSpec((1, SLABW, SLABW), lambda b: (b, 0, 0))],
        out_specs=pl.BlockSpec((1, NSEG, NSEG), lambda b: (b, 0, 0)),
        out_shape=jax.ShapeDtypeStruct((B, NSEG, NSEG), jnp.float32),
    )(cnt)


@jax.jit
def kernel(segments):
    out_flat = _sc_histogram(segments)
    return _symmetrize(out_flat.reshape(B, SLABW, SLABW))


# speedup vs baseline: 2.3852x; 2.3734x over previous
import functools

import numpy as np
import jax
import jax.numpy as jnp
from jax import lax
from jax.experimental import pallas as pl
from jax.experimental.pallas import tpu as pltpu
from jax.experimental.pallas import tpu_sc as plsc

B = 16
H = 512
W = 512
NSEG = 1000
NC = 2
NS = 16
NW = NC * NS
L = 16
SLABW = 1024
SLAB = SLABW * SLABW
RANGE = SLAB // NW
CH = 32
N_CHUNK = H // CH
VPR = W // L
NV = CH * VPR
NLAB = 16000

_GDN = lax.GatherDimensionNumbers(
    offset_dims=(), collapsed_slice_dims=(0,), start_index_map=(0,)
)


def _rot1(v, perm2d):
    return lax.gather(
        v, perm2d, _GDN, (1,), mode=lax.GatherScatterMode.PROMISE_IN_BOUNDS
    )


_INV1000 = np.float32(0.001000000047497451)


def _div1000(x):
    return (x.astype(jnp.float32) * _INV1000).astype(jnp.int32)


def _round_bf16(x_i32):
    u = plsc.bitcast(x_i32.astype(jnp.float32), jnp.int32)
    t = u + 0x7FFF + ((u >> 16) & 1)
    t = t & jnp.int32(-65536)
    return plsc.bitcast(t, jnp.float32).astype(jnp.int32)


def _sc_body(seg_hbm, out_hbm, chunk_v, hist_v):
    c = lax.axis_index("c")
    s = lax.axis_index("s")
    wid = s * NC + c
    lo = wid * RANGE
    ones = jnp.ones((L,), jnp.float32)
    zeros = jnp.zeros((L,), jnp.float32)
    lane = lax.iota(jnp.int32, L)
    perm2d = ((lane + 1) & (L - 1))[:, None]
    lane15 = lane == L - 1
    million = jnp.int32(1_000_000)

    def zero_slot(slot):
        @plsc.parallel_loop(0, RANGE // L, unroll=4)
        def zbody(k):
            hist_v[pl.ds(slot * RANGE + k * L, L)] = zeros

    for slot in range(3):
        zero_slot(slot)

    def batch_body(b, carry):
        off_b = NSEG * b
        slot_prev = (b + 2) % 3
        base_prev = slot_prev * RANGE
        base_cur = (b % 3) * RANGE
        base_next = ((b + 1) % 3) * RANGE
        off_bm1_1000 = NSEG * (b - 1)

        def chunk_body(ci, carry):
            pltpu.sync_copy(seg_hbm.at[b, pl.ds(ci * CH, CH), :], chunk_v)

            @plsc.parallel_loop(0, NV, unroll=4)
            def prep(t):
                r = t >> 5
                j = t & (VPR - 1)
                raw = chunk_v[r, pl.ds(pl.multiple_of(j * L, L), L)]
                xr = _round_bf16(raw + off_b)
                eb1000 = _div1000(xr) * NSEG
                chunk_v[r, pl.ds(pl.multiple_of(j * L, L), L)] = (eb1000 << 14) | xr

            @plsc.parallel_loop(0, NV, unroll=4)
            def inner(t):
                r = t >> 5
                j = t & (VPR - 1)
                p_cur = chunk_v[r, pl.ds(pl.multiple_of(j * L, L), L)]
                tn = jnp.minimum(t + 1, NV - 1)
                rn = tn >> 5
                jn = tn & (VPR - 1)
                p_next = chunk_v[rn, pl.ds(pl.multiple_of(jn * L, L), L)]
                p_d = jnp.where(lane15, _rot1(p_next, perm2d), _rot1(p_cur, perm2d))
                valid = (p_cur != p_d) & ~(lane15 & ((t & (VPR - 1)) == VPR - 1))
                x1 = p_cur & 0x3FFF
                x2 = p_d & 0x3FFF
                eb1000 = p_cur >> 14
                rem0 = (x1 - eb1000) * NSEG + (x2 - eb1000)
                neg = rem0 < 0
                big = rem0 >= million
                rem = rem0 + jnp.where(neg, million, jnp.where(big, -million, 0))
                fi = _div1000(rem)
                fj = rem - fi * NSEG
                reml = (fi << 10) | fj
                d1000 = (eb1000 - off_bm1_1000) + jnp.where(
                    neg, -NSEG, jnp.where(big, NSEG, 0)
                )
                base = jnp.where(
                    d1000 == 0,
                    base_prev,
                    jnp.where(d1000 == NSEG, base_cur, base_next),
                )
                u = reml - lo
                m = valid & (u >= 0) & (u < RANGE)
                plsc.addupdate_scatter(hist_v, [base + u], ones, mask=m)

            return carry

        lax.fori_loop(0, N_CHUNK, chunk_body, 0)

        @pl.when(b >= 1)
        def _flush():
            off = pl.multiple_of((b - 1) * SLAB + lo, 8)
            pltpu.sync_copy(
                hist_v.at[pl.ds(base_prev, RANGE)], out_hbm.at[pl.ds(off, RANGE)]
            )
            zero_slot(slot_prev)

        return carry

    lax.fori_loop(0, B, batch_body, 0, unroll=3)
    off = pl.multiple_of((B - 1) * SLAB + lo, 8)
    pltpu.sync_copy(hist_v.at[pl.ds(0, RANGE)], out_hbm.at[pl.ds(off, RANGE)])


def _sc_histogram(segments):
    mesh = plsc.VectorSubcoreMesh(
        core_axis_name="c", subcore_axis_name="s", num_cores=NC, num_subcores=NS
    )
    return pl.kernel(
        _sc_body,
        out_type=jax.ShapeDtypeStruct((B * SLAB,), jnp.float32),
        mesh=mesh,
        scratch_types=[
            pltpu.VMEM((CH, W), jnp.int32),
            pltpu.VMEM((3 * RANGE,), jnp.float32),
        ],
        compiler_params=pltpu.CompilerParams(needs_layout_passes=False),
    )(segments)


def _sym_body(x_ref, o_ref):
    x = x_ref[0]
    y = x + x.T
    o_ref[0] = y[:NSEG, :NSEG]


def _symmetrize(cnt):
    return pl.pallas_call(
        _sym_body,
        grid=(B,),
        in_specs=[pl.BlockSpec((1, SLABW, SLABW), lambda b: (b, 0, 0))],
        out_specs=pl.BlockSpec((1, NSEG, NSEG), lambda b: (b, 0, 0)),
        out_shape=jax.ShapeDtypeStruct((B, NSEG, NSEG), jnp.float32),
    )(cnt)


@jax.jit
def kernel(segments):
    out_flat = _sc_histogram(segments)
    return _symmetrize(out_flat.reshape(B, SLABW, SLABW))


# speedup vs baseline: 2.4681x; 1.0347x over previous
import functools

import numpy as np
import jax
import jax.numpy as jnp
from jax import lax
from jax.experimental import pallas as pl
from jax.experimental.pallas import tpu as pltpu
from jax.experimental.pallas import tpu_sc as plsc

B = 16
H = 512
W = 512
NSEG = 1000
NC = 2
NS = 16
NW = NC * NS
L = 16
SLABW = 1024
SLAB = SLABW * SLABW
RANGE = SLAB // NW
CH = 32
N_CHUNK = H // CH
VPR = W // L
NV = CH * VPR
NLAB = 16000

_GDN = lax.GatherDimensionNumbers(
    offset_dims=(), collapsed_slice_dims=(0,), start_index_map=(0,)
)


def _rot1(v, perm2d):
    return lax.gather(
        v, perm2d, _GDN, (1,), mode=lax.GatherScatterMode.PROMISE_IN_BOUNDS
    )


_INV1000 = np.float32(0.001000000047497451)


def _div1000(x):
    return (x.astype(jnp.float32) * _INV1000).astype(jnp.int32)


def _round_bf16(x_i32):
    u = plsc.bitcast(x_i32.astype(jnp.float32), jnp.int32)
    t = u + 0x7FFF + ((u >> 16) & 1)
    t = t & jnp.int32(-65536)
    return plsc.bitcast(t, jnp.float32).astype(jnp.int32)


def _sc_body(seg_hbm, out_hbm, chunk_v, hist_v):
    c = lax.axis_index("c")
    s = lax.axis_index("s")
    wid = s * NC + c
    lo = wid * RANGE
    ones = jnp.ones((L,), jnp.float32)
    zeros = jnp.zeros((L,), jnp.float32)
    lane = lax.iota(jnp.int32, L)
    perm2d = ((lane + 1) & (L - 1))[:, None]
    lane15 = lane == L - 1
    not_lane15 = ~lane15
    all_true = lane >= 0
    million = jnp.int32(1_000_000)
    urange = jnp.uint32(RANGE)

    def zero_slot(slot):
        @plsc.parallel_loop(0, RANGE // L, unroll=4)
        def zbody(k):
            hist_v[pl.ds(slot * RANGE + k * L, L)] = zeros

    for slot in range(3):
        zero_slot(slot)

    def batch_body(b, carry):
        off_b = NSEG * b
        slot_prev = (b + 2) % 3
        base_prev = slot_prev * RANGE
        base_cur = (b % 3) * RANGE
        base_next = ((b + 1) % 3) * RANGE
        off_bm1_1000 = NSEG * (b - 1)

        def chunk_body(ci, carry):
            pltpu.sync_copy(seg_hbm.at[b, pl.ds(ci * CH, CH), :], chunk_v)

            @plsc.parallel_loop(0, NV, unroll=4)
            def prep(t):
                r = t >> 5
                j = t & (VPR - 1)
                raw = chunk_v[r, pl.ds(pl.multiple_of(j * L, L), L)]
                xr = _round_bf16(raw + off_b)
                eb1000 = _div1000(xr) * NSEG
                chunk_v[r, pl.ds(pl.multiple_of(j * L, L), L)] = (eb1000 << 14) | xr

            @plsc.parallel_loop(0, NV, unroll=4)
            def inner(t):
                r = t >> 5
                j = t & (VPR - 1)
                p_cur = chunk_v[r, pl.ds(pl.multiple_of(j * L, L), L)]
                tn = jnp.minimum(t + 1, NV - 1)
                rn = tn >> 5
                jn = tn & (VPR - 1)
                p_next = chunk_v[rn, pl.ds(pl.multiple_of(jn * L, L), L)]
                p_d = jnp.where(lane15, _rot1(p_next, perm2d), _rot1(p_cur, perm2d))
                nl = jnp.where((t & (VPR - 1)) == VPR - 1, not_lane15, all_true)
                valid = (p_cur != p_d) & nl
                x1 = p_cur & 0x3FFF
                x2 = p_d & 0x3FFF
                eb1000 = p_cur >> 14
                rem0 = (x1 - eb1000) * NSEG + (x2 - eb1000)
                neg = rem0 < 0
                big = rem0 >= million
                rem = rem0 + jnp.where(neg, million, jnp.where(big, -million, 0))
                fi = _div1000(rem)
                reml = rem + 24 * fi
                d1000 = (eb1000 - off_bm1_1000) + jnp.where(
                    neg, -NSEG, jnp.where(big, NSEG, 0)
                )
                base = jnp.where(
                    d1000 == 0,
                    base_prev,
                    jnp.where(d1000 == NSEG, base_cur, base_next),
                )
                u = reml - lo
                m = valid & (plsc.bitcast(u, jnp.uint32) < urange)
                plsc.addupdate_scatter(hist_v, [base + u], ones, mask=m)

            return carry

        lax.fori_loop(0, N_CHUNK, chunk_body, 0)

        @pl.when(b >= 1)
        def _flush():
            off = pl.multiple_of((b - 1) * SLAB + lo, 8)
            pltpu.sync_copy(
                hist_v.at[pl.ds(base_prev, RANGE)], out_hbm.at[pl.ds(off, RANGE)]
            )
            zero_slot(slot_prev)

        return carry

    lax.fori_loop(0, B, batch_body, 0, unroll=3)
    off = pl.multiple_of((B - 1) * SLAB + lo, 8)
    pltpu.sync_copy(hist_v.at[pl.ds(0, RANGE)], out_hbm.at[pl.ds(off, RANGE)])


def _sc_histogram(segments):
    mesh = plsc.VectorSubcoreMesh(
        core_axis_name="c", subcore_axis_name="s", num_cores=NC, num_subcores=NS
    )
    return pl.kernel(
        _sc_body,
        out_type=jax.ShapeDtypeStruct((B * SLAB,), jnp.float32),
        mesh=mesh,
        scratch_types=[
            pltpu.VMEM((CH, W), jnp.int32),
            pltpu.VMEM((3 * RANGE,), jnp.float32),
        ],
        compiler_params=pltpu.CompilerParams(needs_layout_passes=False),
    )(segments)


def _sym_body(x_ref, o_ref):
    x = x_ref[0]
    y = x + x.T
    o_ref[0] = y[:NSEG, :NSEG]


def _symmetrize(cnt):
    return pl.pallas_call(
        _sym_body,
        grid=(B,),
        in_specs=[pl.BlockSpec((1, SLABW, SLABW), lambda b: (b, 0, 0))],
        out_specs=pl.BlockSpec((1, NSEG, NSEG), lambda b: (b, 0, 0)),
        out_shape=jax.ShapeDtypeStruct((B, NSEG, NSEG), jnp.float32),
    )(cnt)


@jax.jit
def kernel(segments):
    out_flat = _sc_histogram(segments)
    return _symmetrize(out_flat.reshape(B, SLABW, SLABW))


# speedup vs baseline: 2.4972x; 1.0118x over previous
import functools

import numpy as np
import jax
import jax.numpy as jnp
from jax import lax
from jax.experimental import pallas as pl
from jax.experimental.pallas import tpu as pltpu
from jax.experimental.pallas import tpu_sc as plsc

B = 16
H = 512
W = 512
NSEG = 1000
NC = 2
NS = 16
NW = NC * NS
L = 16
SLABW = 1024
SLAB = SLABW * SLABW
RANGE = SLAB // NW
CH = 32
N_CHUNK = H // CH
VPR = W // L
NV = CH * VPR
NLAB = 16000

_GDN = lax.GatherDimensionNumbers(
    offset_dims=(), collapsed_slice_dims=(0,), start_index_map=(0,)
)


def _rot1(v, perm2d):
    return lax.gather(
        v, perm2d, _GDN, (1,), mode=lax.GatherScatterMode.PROMISE_IN_BOUNDS
    )


_INV1000 = np.float32(0.001000000047497451)


def _div1000(x):
    return (x.astype(jnp.float32) * _INV1000).astype(jnp.int32)


def _round_bf16(x_i32):
    u = plsc.bitcast(x_i32.astype(jnp.float32), jnp.int32)
    t = u + 0x7FFF + ((u >> 16) & 1)
    t = t & jnp.int32(-65536)
    return plsc.bitcast(t, jnp.float32).astype(jnp.int32)


def _sc_body(seg_hbm, out_hbm, chunk_v, hist_v):
    c = lax.axis_index("c")
    s = lax.axis_index("s")
    wid = s * NC + c
    lo = wid * RANGE
    ones = jnp.ones((L,), jnp.float32)
    zeros = jnp.zeros((L,), jnp.float32)
    lane = lax.iota(jnp.int32, L)
    perm2d = ((lane + 1) & (L - 1))[:, None]
    lane15 = lane == L - 1
    not_lane15 = ~lane15
    all_true = lane >= 0
    million = jnp.int32(1_000_000)
    urange = jnp.uint32(RANGE)

    def zero_slot(slot):
        @plsc.parallel_loop(0, RANGE // L, unroll=4)
        def zbody(k):
            hist_v[pl.ds(slot * RANGE + k * L, L)] = zeros

    for slot in range(3):
        zero_slot(slot)

    def batch_body(b, carry):
        off_b = NSEG * b
        slot_prev = (b + 2) % 3
        base_prev = slot_prev * RANGE
        base_cur = (b % 3) * RANGE
        base_next = ((b + 1) % 3) * RANGE
        off_bm1_1000 = NSEG * (b - 1)

        def chunk_body(ci, carry):
            pltpu.sync_copy(seg_hbm.at[b, pl.ds(ci * CH, CH), :], chunk_v)

            @plsc.parallel_loop(0, NV, unroll=4)
            def prep(t):
                r = t >> 5
                j = t & (VPR - 1)
                raw = chunk_v[r, pl.ds(pl.multiple_of(j * L, L), L)]
                xr = _round_bf16(raw + off_b)
                eb1000 = _div1000(xr) * NSEG
                chunk_v[r, pl.ds(pl.multiple_of(j * L, L), L)] = (eb1000 << 14) | xr

            @plsc.parallel_loop(0, NV, unroll=8)
            def inner(t):
                r = t >> 5
                j = t & (VPR - 1)
                p_cur = chunk_v[r, pl.ds(pl.multiple_of(j * L, L), L)]
                tn = jnp.minimum(t + 1, NV - 1)
                rn = tn >> 5
                jn = tn & (VPR - 1)
                p_next = chunk_v[rn, pl.ds(pl.multiple_of(jn * L, L), L)]
                p_d = jnp.where(lane15, _rot1(p_next, perm2d), _rot1(p_cur, perm2d))
                nl = jnp.where((t & (VPR - 1)) == VPR - 1, not_lane15, all_true)
                valid = (p_cur != p_d) & nl
                x1 = p_cur & 0x3FFF
                x2 = p_d & 0x3FFF
                eb1000 = p_cur >> 14
                rem0 = (x1 - eb1000) * NSEG + (x2 - eb1000)
                neg = rem0 < 0
                big = rem0 >= million
                rem = rem0 + jnp.where(neg, million, jnp.where(big, -million, 0))
                fi = _div1000(rem)
                reml = rem + 24 * fi
                d1000 = (eb1000 - off_bm1_1000) + jnp.where(
                    neg, -NSEG, jnp.where(big, NSEG, 0)
                )
                base = jnp.where(
                    d1000 == 0,
                    base_prev,
                    jnp.where(d1000 == NSEG, base_cur, base_next),
                )
                u = reml - lo
                m = valid & (plsc.bitcast(u, jnp.uint32) < urange)
                plsc.addupdate_scatter(hist_v, [base + u], ones, mask=m)

            return carry

        lax.fori_loop(0, N_CHUNK, chunk_body, 0)

        @pl.when(b >= 1)
        def _flush():
            off = pl.multiple_of((b - 1) * SLAB + lo, 8)
            pltpu.sync_copy(
                hist_v.at[pl.ds(base_prev, RANGE)], out_hbm.at[pl.ds(off, RANGE)]
            )
            zero_slot(slot_prev)

        return carry

    lax.fori_loop(0, B, batch_body, 0, unroll=3)
    off = pl.multiple_of((B - 1) * SLAB + lo, 8)
    pltpu.sync_copy(hist_v.at[pl.ds(0, RANGE)], out_hbm.at[pl.ds(off, RANGE)])


def _sc_histogram(segments):
    mesh = plsc.VectorSubcoreMesh(
        core_axis_name="c", subcore_axis_name="s", num_cores=NC, num_subcores=NS
    )
    return pl.kernel(
        _sc_body,
        out_type=jax.ShapeDtypeStruct((B * SLAB,), jnp.float32),
        mesh=mesh,
        scratch_types=[
            pltpu.VMEM((CH, W), jnp.int32),
            pltpu.VMEM((3 * RANGE,), jnp.float32),
        ],
        compiler_params=pltpu.CompilerParams(needs_layout_passes=False),
    )(segments)


def _sym_body(x_ref, o_ref):
    x = x_ref[0]
    y = x + x.T
    o_ref[0] = y[:NSEG, :NSEG]


def _symmetrize(cnt):
    return pl.pallas_call(
        _sym_body,
        grid=(B,),
        in_specs=[pl.BlockSpec((1, SLABW, SLABW), lambda b: (b, 0, 0))],
        out_specs=pl.BlockSpec((1, NSEG, NSEG), lambda b: (b, 0, 0)),
        out_shape=jax.ShapeDtypeStruct((B, NSEG, NSEG), jnp.float32),
    )(cnt)


@jax.jit
def kernel(segments):
    out_flat = _sc_histogram(segments)
    return _symmetrize(out_flat.reshape(B, SLABW, SLABW))


# speedup vs baseline: 2.7521x; 1.1021x over previous
import functools

import numpy as np
import jax
import jax.numpy as jnp
from jax import lax
from jax.experimental import pallas as pl
from jax.experimental.pallas import tpu as pltpu
from jax.experimental.pallas import tpu_sc as plsc

B = 16
H = 512
W = 512
NSEG = 1000
NC = 2
NS = 16
NW = NC * NS
L = 16
SLABW = 1024
SLAB = SLABW * SLABW
RANGE = SLAB // NW
CH = 16
N_CHUNK = H // CH
VPR = W // L
NV = CH * VPR
NLAB = 16000

_GDN = lax.GatherDimensionNumbers(
    offset_dims=(), collapsed_slice_dims=(0,), start_index_map=(0,)
)


def _rot1(v, perm2d):
    return lax.gather(
        v, perm2d, _GDN, (1,), mode=lax.GatherScatterMode.PROMISE_IN_BOUNDS
    )


_INV1000 = np.float32(0.001000000047497451)


def _div1000(x):
    return (x.astype(jnp.float32) * _INV1000).astype(jnp.int32)


def _round_bf16(x_i32):
    u = plsc.bitcast(x_i32.astype(jnp.float32), jnp.int32)
    t = u + 0x7FFF + ((u >> 16) & 1)
    t = t & jnp.int32(-65536)
    return plsc.bitcast(t, jnp.float32).astype(jnp.int32)


def _sc_body(seg_hbm, out_hbm, chunk_v, hist_v, sem0, sem1):
    c = lax.axis_index("c")
    s = lax.axis_index("s")
    wid = s * NC + c
    lo = wid * RANGE
    ones = jnp.ones((L,), jnp.float32)
    zeros = jnp.zeros((L,), jnp.float32)
    lane = lax.iota(jnp.int32, L)
    perm2d = ((lane + 1) & (L - 1))[:, None]
    lane15 = lane == L - 1
    not_lane15 = ~lane15
    all_true = lane >= 0
    million = jnp.int32(1_000_000)
    urange = jnp.uint32(RANGE)

    def zero_slot(slot):
        @plsc.parallel_loop(0, RANGE // L, unroll=4)
        def zbody(k):
            hist_v[pl.ds(slot * RANGE + k * L, L)] = zeros

    for slot in range(3):
        zero_slot(slot)

    def batch_body(b, carry):
        off_b = NSEG * b
        slot_prev = (b + 2) % 3
        base_prev = slot_prev * RANGE
        base_cur = (b % 3) * RANGE
        base_next = ((b + 1) % 3) * RANGE
        off_bm1_1000 = NSEG * (b - 1)

        def dma(ci, buf, sem):
            return pltpu.make_async_copy(
                seg_hbm.at[b, pl.ds(ci * CH, CH), :], chunk_v.at[buf], sem
            )

        def process(ch_ref):
            @plsc.parallel_loop(0, NV, unroll=4)
            def prep(t):
                r = t >> 5
                j = t & (VPR - 1)
                raw = ch_ref[r, pl.ds(pl.multiple_of(j * L, L), L)]
                xr = _round_bf16(raw + off_b)
                eb1000 = _div1000(xr) * NSEG
                ch_ref[r, pl.ds(pl.multiple_of(j * L, L), L)] = (eb1000 << 14) | xr

            @plsc.parallel_loop(0, NV, unroll=8)
            def inner(t):
                r = t >> 5
                j = t & (VPR - 1)
                p_cur = ch_ref[r, pl.ds(pl.multiple_of(j * L, L), L)]
                tn = jnp.minimum(t + 1, NV - 1)
                rn = tn >> 5
                jn = tn & (VPR - 1)
                p_next = ch_ref[rn, pl.ds(pl.multiple_of(jn * L, L), L)]
                p_d = jnp.where(lane15, _rot1(p_next, perm2d), _rot1(p_cur, perm2d))
                nl = jnp.where((t & (VPR - 1)) == VPR - 1, not_lane15, all_true)
                valid = (p_cur != p_d) & nl
                x1 = p_cur & 0x3FFF
                x2 = p_d & 0x3FFF
                eb1000 = p_cur >> 14
                rem0 = (x1 - eb1000) * NSEG + (x2 - eb1000)
                neg = rem0 < 0
                big = rem0 >= million
                rem = rem0 + jnp.where(neg, million, jnp.where(big, -million, 0))
                fi = _div1000(rem)
                reml = rem + 24 * fi
                d1000 = (eb1000 - off_bm1_1000) + jnp.where(
                    neg, -NSEG, jnp.where(big, NSEG, 0)
                )
                base = jnp.where(
                    d1000 == 0,
                    base_prev,
                    jnp.where(d1000 == NSEG, base_cur, base_next),
                )
                u = reml - lo
                m = valid & (plsc.bitcast(u, jnp.uint32) < urange)
                plsc.addupdate_scatter(hist_v, [base + u], ones, mask=m)

        dma(0, 0, sem0).start()

        def pair_body(cp, carry):
            ci = cp * 2
            dma(ci + 1, 1, sem1).start()
            dma(ci, 0, sem0).wait()
            process(chunk_v.at[0])

            @pl.when(cp < N_CHUNK // 2 - 1)
            def _prefetch():
                dma(ci + 2, 0, sem0).start()

            dma(ci + 1, 1, sem1).wait()
            process(chunk_v.at[1])
            return carry

        lax.fori_loop(0, N_CHUNK // 2, pair_body, 0)

        @pl.when(b >= 1)
        def _flush():
            off = pl.multiple_of((b - 1) * SLAB + lo, 8)
            pltpu.sync_copy(
                hist_v.at[pl.ds(base_prev, RANGE)], out_hbm.at[pl.ds(off, RANGE)]
            )
            zero_slot(slot_prev)

        return carry

    lax.fori_loop(0, B, batch_body, 0, unroll=3)
    off = pl.multiple_of((B - 1) * SLAB + lo, 8)
    pltpu.sync_copy(hist_v.at[pl.ds(0, RANGE)], out_hbm.at[pl.ds(off, RANGE)])


def _sc_histogram(segments):
    mesh = plsc.VectorSubcoreMesh(
        core_axis_name="c", subcore_axis_name="s", num_cores=NC, num_subcores=NS
    )
    return pl.kernel(
        _sc_body,
        out_type=jax.ShapeDtypeStruct((B * SLAB,), jnp.float32),
        mesh=mesh,
        scratch_types=[
            pltpu.VMEM((2, CH, W), jnp.int32),
            pltpu.VMEM((3 * RANGE,), jnp.float32),
            pltpu.SemaphoreType.DMA,
            pltpu.SemaphoreType.DMA,
        ],
        compiler_params=pltpu.CompilerParams(needs_layout_passes=False),
    )(segments)


def _sym_body(x_ref, o_ref):
    x = x_ref[0]
    y = x + x.T
    o_ref[0] = y[:NSEG, :NSEG]


def _symmetrize(cnt):
    return pl.pallas_call(
        _sym_body,
        grid=(B,),
        in_specs=[pl.BlockSpec((1, SLABW, SLABW), lambda b: (b, 0, 0))],
        out_specs=pl.BlockSpec((1, NSEG, NSEG), lambda b: (b, 0, 0)),
        out_shape=jax.ShapeDtypeStruct((B, NSEG, NSEG), jnp.float32),
    )(cnt)


@jax.jit
def kernel(segments):
    out_flat = _sc_histogram(segments)
    return _symmetrize(out_flat.reshape(B, SLABW, SLABW))


# speedup vs baseline: 2.7527x; 1.0002x over previous
import functools

import numpy as np
import jax
import jax.numpy as jnp
from jax import lax
from jax.experimental import pallas as pl
from jax.experimental.pallas import tpu as pltpu
from jax.experimental.pallas import tpu_sc as plsc

B = 16
H = 512
W = 512
NSEG = 1000
NC = 2
NS = 16
NW = NC * NS
L = 16
SLABW = 1024
SLAB = SLABW * SLABW
RANGE = SLAB // NW
CH = 16
N_CHUNK = H // CH
VPR = W // L
NV = CH * VPR
NLAB = 16000

_GDN = lax.GatherDimensionNumbers(
    offset_dims=(), collapsed_slice_dims=(0,), start_index_map=(0,)
)


def _rot1(v, perm2d):
    return lax.gather(
        v, perm2d, _GDN, (1,), mode=lax.GatherScatterMode.PROMISE_IN_BOUNDS
    )


_INV1000 = np.float32(0.001000000047497451)


def _div1000(x):
    return (x.astype(jnp.float32) * _INV1000).astype(jnp.int32)


def _round_bf16(x_i32):
    u = plsc.bitcast(x_i32.astype(jnp.float32), jnp.int32)
    t = u + 0x7FFF + ((u >> 16) & 1)
    t = t & jnp.int32(-65536)
    return plsc.bitcast(t, jnp.float32).astype(jnp.int32)


def _sc_body(seg_hbm, out_hbm, chunk_v, hist_v, sem0, sem1):
    c = lax.axis_index("c")
    s = lax.axis_index("s")
    wid = s * NC + c
    lo = wid * RANGE
    ones = jnp.ones((L,), jnp.float32)
    zeros = jnp.zeros((L,), jnp.float32)
    lane = lax.iota(jnp.int32, L)
    perm2d = ((lane + 1) & (L - 1))[:, None]
    lane15 = lane == L - 1
    not_lane15 = ~lane15
    all_true = lane >= 0
    million = jnp.int32(1_000_000)
    urange = jnp.uint32(RANGE)

    def zero_slot(slot):
        @plsc.parallel_loop(0, RANGE // L, unroll=4)
        def zbody(k):
            hist_v[pl.ds(slot * RANGE + k * L, L)] = zeros

    for slot in range(3):
        zero_slot(slot)

    def batch_body(b, carry):
        off_b = NSEG * b
        slot_prev = (b + 2) % 3
        base_prev = slot_prev * RANGE
        base_cur = (b % 3) * RANGE
        base_next = ((b + 1) % 3) * RANGE
        off_bm1_1000 = NSEG * (b - 1)

        def dma(ci, buf, sem):
            return pltpu.make_async_copy(
                seg_hbm.at[b, pl.ds(ci * CH, CH), :], chunk_v.at[buf], sem
            )

        def process(ch_ref):
            @plsc.parallel_loop(0, NV, unroll=8)
            def prep(t):
                r = t >> 5
                j = t & (VPR - 1)
                raw = ch_ref[r, pl.ds(pl.multiple_of(j * L, L), L)]
                xr = _round_bf16(raw + off_b)
                eb1000 = _div1000(xr) * NSEG
                ch_ref[r, pl.ds(pl.multiple_of(j * L, L), L)] = (eb1000 << 14) | xr

            @plsc.parallel_loop(0, NV, unroll=8)
            def inner(t):
                r = t >> 5
                j = t & (VPR - 1)
                p_cur = ch_ref[r, pl.ds(pl.multiple_of(j * L, L), L)]
                tn = jnp.minimum(t + 1, NV - 1)
                rn = tn >> 5
                jn = tn & (VPR - 1)
                p_next = ch_ref[rn, pl.ds(pl.multiple_of(jn * L, L), L)]
                p_d = jnp.where(lane15, _rot1(p_next, perm2d), _rot1(p_cur, perm2d))
                nl = jnp.where((t & (VPR - 1)) == VPR - 1, not_lane15, all_true)
                valid = (p_cur != p_d) & nl
                x1 = p_cur & 0x3FFF
                x2 = p_d & 0x3FFF
                eb1000 = p_cur >> 14
                rem0 = (x1 - eb1000) * NSEG + (x2 - eb1000)
                neg = rem0 < 0
                big = rem0 >= million
                rem = rem0 + jnp.where(neg, million, jnp.where(big, -million, 0))
                fi = _div1000(rem)
                reml = rem + 24 * fi
                d1000 = (eb1000 - off_bm1_1000) + jnp.where(
                    neg, -NSEG, jnp.where(big, NSEG, 0)
                )
                base = jnp.where(
                    d1000 == 0,
                    base_prev,
                    jnp.where(d1000 == NSEG, base_cur, base_next),
                )
                u = reml - lo
                m = valid & (plsc.bitcast(u, jnp.uint32) < urange)
                plsc.addupdate_scatter(hist_v, [base + u], ones, mask=m)

        dma(0, 0, sem0).start()

        def pair_body(cp, carry):
            ci = cp * 2
            dma(ci + 1, 1, sem1).start()
            dma(ci, 0, sem0).wait()
            process(chunk_v.at[0])

            @pl.when(cp < N_CHUNK // 2 - 1)
            def _prefetch():
                dma(ci + 2, 0, sem0).start()

            dma(ci + 1, 1, sem1).wait()
            process(chunk_v.at[1])
            return carry

        lax.fori_loop(0, N_CHUNK // 2, pair_body, 0)

        @pl.when(b >= 1)
        def _flush():
            off = pl.multiple_of((b - 1) * SLAB + lo, 8)
            pltpu.sync_copy(
                hist_v.at[pl.ds(base_prev, RANGE)], out_hbm.at[pl.ds(off, RANGE)]
            )
            zero_slot(slot_prev)

        return carry

    lax.fori_loop(0, B, batch_body, 0, unroll=3)
    off = pl.multiple_of((B - 1) * SLAB + lo, 8)
    pltpu.sync_copy(hist_v.at[pl.ds(0, RANGE)], out_hbm.at[pl.ds(off, RANGE)])


def _sc_histogram(segments):
    mesh = plsc.VectorSubcoreMesh(
        core_axis_name="c", subcore_axis_name="s", num_cores=NC, num_subcores=NS
    )
    return pl.kernel(
        _sc_body,
        out_type=jax.ShapeDtypeStruct((B * SLAB,), jnp.float32),
        mesh=mesh,
        scratch_types=[
            pltpu.VMEM((2, CH, W), jnp.int32),
            pltpu.VMEM((3 * RANGE,), jnp.float32),
            pltpu.SemaphoreType.DMA,
            pltpu.SemaphoreType.DMA,
        ],
        compiler_params=pltpu.CompilerParams(needs_layout_passes=False),
    )(segments)


def _sym_body(x_ref, o_ref):
    x = x_ref[0]
    y = x + x.T
    o_ref[0] = y[:NSEG, :NSEG]


def _symmetrize(cnt):
    return pl.pallas_call(
        _sym_body,
        grid=(B,),
        in_specs=[pl.BlockSpec((1, SLABW, SLABW), lambda b: (b, 0, 0))],
        out_specs=pl.BlockSpec((1, NSEG, NSEG), lambda b: (b, 0, 0)),
        out_shape=jax.ShapeDtypeStruct((B, NSEG, NSEG), jnp.float32),
    )(cnt)


@jax.jit
def kernel(segments):
    out_flat = _sc_histogram(segments)
    return _symmetrize(out_flat.reshape(B, SLABW, SLABW))


# speedup vs baseline: 2.7528x; 1.0000x over previous
import numpy as np
import jax
import jax.numpy as jnp
from jax import lax
from jax.experimental import pallas as pl
from jax.experimental.pallas import tpu as pltpu
from jax.experimental.pallas import tpu_sc as plsc

B = 16
H = 512
W = 512
NSEG = 1000
NC = 2
NS = 16
NW = NC * NS
L = 16
SLABW = 1024
SLAB = SLABW * SLABW
RANGE = SLAB // NW
CH = 16
N_CHUNK = H // CH
VPR = W // L
NV = CH * VPR

_GDN = lax.GatherDimensionNumbers(
    offset_dims=(), collapsed_slice_dims=(0,), start_index_map=(0,)
)


def _rot1(v, perm2d):
    return lax.gather(
        v, perm2d, _GDN, (1,), mode=lax.GatherScatterMode.PROMISE_IN_BOUNDS
    )


_INV1000 = np.float32(0.001000000047497451)


def _div1000(x):
    return (x.astype(jnp.float32) * _INV1000).astype(jnp.int32)


def _round_bf16(x_i32):
    u = plsc.bitcast(x_i32.astype(jnp.float32), jnp.int32)
    t = u + 0x7FFF + ((u >> 16) & 1)
    t = t & jnp.int32(-65536)
    return plsc.bitcast(t, jnp.float32).astype(jnp.int32)


def _sc_body(seg_hbm, out_hbm, chunk_v, hist_v, sem0, sem1):
    c = lax.axis_index("c")
    s = lax.axis_index("s")
    wid = s * NC + c
    lo = wid * RANGE
    ones = jnp.ones((L,), jnp.float32)
    zeros = jnp.zeros((L,), jnp.float32)
    lane = lax.iota(jnp.int32, L)
    perm2d = ((lane + 1) & (L - 1))[:, None]
    lane15 = lane == L - 1
    not_lane15 = ~lane15
    all_true = lane >= 0
    million = jnp.int32(1_000_000)
    urange = jnp.uint32(RANGE)

    def zero_slot(slot):
        @plsc.parallel_loop(0, RANGE // L, unroll=4)
        def zbody(k):
            hist_v[pl.ds(slot * RANGE + k * L, L)] = zeros

    for slot in range(3):
        zero_slot(slot)

    def batch_body(b, carry):
        off_b = NSEG * b
        slot_prev = (b + 2) % 3
        base_prev = slot_prev * RANGE
        base_cur = (b % 3) * RANGE
        base_next = ((b + 1) % 3) * RANGE
        off_bm1_1000 = NSEG * (b - 1)

        def dma(ci, buf, sem):
            return pltpu.make_async_copy(
                seg_hbm.at[b, pl.ds(ci * CH, CH), :], chunk_v.at[buf], sem
            )

        def process(ch_ref):
            @plsc.parallel_loop(0, NV, unroll=8)
            def prep(t):
                r = t >> 5
                j = t & (VPR - 1)
                raw = ch_ref[r, pl.ds(pl.multiple_of(j * L, L), L)]
                xr = _round_bf16(raw + off_b)
                eb1000 = _div1000(xr) * NSEG
                ch_ref[r, pl.ds(pl.multiple_of(j * L, L), L)] = (eb1000 << 14) | xr

            @plsc.parallel_loop(0, NV, unroll=8)
            def inner(t):
                r = t >> 5
                j = t & (VPR - 1)
                p_cur = ch_ref[r, pl.ds(pl.multiple_of(j * L, L), L)]
                tn = jnp.minimum(t + 1, NV - 1)
                rn = tn >> 5
                jn = tn & (VPR - 1)
                p_next = ch_ref[rn, pl.ds(pl.multiple_of(jn * L, L), L)]
                p_d = jnp.where(lane15, _rot1(p_next, perm2d), _rot1(p_cur, perm2d))
                nl = jnp.where((t & (VPR - 1)) == VPR - 1, not_lane15, all_true)
                valid = (p_cur != p_d) & nl
                x1 = p_cur & 0x3FFF
                x2 = p_d & 0x3FFF
                eb1000 = p_cur >> 14
                rem0 = (x1 - eb1000) * NSEG + (x2 - eb1000)
                neg = rem0 < 0
                big = rem0 >= million
                rem = rem0 + jnp.where(neg, million, jnp.where(big, -million, 0))
                fi = _div1000(rem)
                reml = rem + 24 * fi
                d1000 = (eb1000 - off_bm1_1000) + jnp.where(
                    neg, -NSEG, jnp.where(big, NSEG, 0)
                )
                base = jnp.where(
                    d1000 == 0,
                    base_prev,
                    jnp.where(d1000 == NSEG, base_cur, base_next),
                )
                u = reml - lo
                m = valid & (plsc.bitcast(u, jnp.uint32) < urange)
                plsc.addupdate_scatter(hist_v, [base + u], ones, mask=m)

        dma(0, 0, sem0).start()

        def pair_body(cp, carry):
            ci = cp * 2
            dma(ci + 1, 1, sem1).start()
            dma(ci, 0, sem0).wait()
            process(chunk_v.at[0])

            @pl.when(cp < N_CHUNK // 2 - 1)
            def _prefetch():
                dma(ci + 2, 0, sem0).start()

            dma(ci + 1, 1, sem1).wait()
            process(chunk_v.at[1])
            return carry

        lax.fori_loop(0, N_CHUNK // 2, pair_body, 0)

        @pl.when(b >= 1)
        def _flush():
            off = pl.multiple_of((b - 1) * SLAB + lo, 8)
            pltpu.sync_copy(
                hist_v.at[pl.ds(base_prev, RANGE)], out_hbm.at[pl.ds(off, RANGE)]
            )
            zero_slot(slot_prev)

        return carry

    lax.fori_loop(0, B, batch_body, 0, unroll=3)
    off = pl.multiple_of((B - 1) * SLAB + lo, 8)
    pltpu.sync_copy(hist_v.at[pl.ds(0, RANGE)], out_hbm.at[pl.ds(off, RANGE)])


def _sc_histogram(segments):
    mesh = plsc.VectorSubcoreMesh(
        core_axis_name="c", subcore_axis_name="s", num_cores=NC, num_subcores=NS
    )
    return pl.kernel(
        _sc_body,
        out_type=jax.ShapeDtypeStruct((B * SLAB,), jnp.float32),
        mesh=mesh,
        scratch_types=[
            pltpu.VMEM((2, CH, W), jnp.int32),
            pltpu.VMEM((3 * RANGE,), jnp.float32),
            pltpu.SemaphoreType.DMA,
            pltpu.SemaphoreType.DMA,
        ],
        compiler_params=pltpu.CompilerParams(needs_layout_passes=False),
    )(segments)


def _sym_body(x_ref, o_ref):
    x = x_ref[0]
    y = x + x.T
    o_ref[0] = y[:NSEG, :NSEG]


def _symmetrize(cnt):
    return pl.pallas_call(
        _sym_body,
        grid=(B,),
        in_specs=[pl.BlockSpec((1, SLABW, SLABW), lambda b: (b, 0, 0))],
        out_specs=pl.BlockSpec((1, NSEG, NSEG), lambda b: (b, 0, 0)),
        out_shape=jax.ShapeDtypeStruct((B, NSEG, NSEG), jnp.float32),
    )(cnt)


@jax.jit
def kernel(segments):
    out_flat = _sc_histogram(segments)
    return _symmetrize(out_flat.reshape(B, SLABW, SLABW))
